# baseline (speedup 1.0000x reference)
import functools

import jax
import jax.numpy as jnp
from jax import lax
from jax.experimental import pallas as pl
from jax.experimental.pallas import tpu as pltpu
from jax.experimental.pallas import tpu_sc as plsc

N = 10000
E = 320000
D = 128
R = 8

NC = 2
NS = 16
NW = NC * NS
L = 16

C = 80
NCHUNK = E // C
CH_PER_W = NCHUNK // NW
EPW = CH_PER_W * C

ROWS_PER_TILE = 624
TAIL_ROW0 = NS * ROWS_PER_TILE
TAIL_ROWS = N - TAIL_ROW0

_BN = 1000


DST_BITS = 14
_E2 = E // 128


def _proj_pack_body(x_ref, w_ref, src_ref, et_ref, dst_ref,
                    proj_ref, packed_ref):
    xb = x_ref[...]
    for r in range(R):
        proj_ref[:, r, :] = jnp.dot(xb, w_ref[r],
                                    preferred_element_type=jnp.float32)

    @pl.when(pl.program_id(0) == 0)
    def _pack():
        idx = src_ref[...] * R + et_ref[...]
        packed_ref[...] = jnp.left_shift(idx, DST_BITS) + dst_ref[...]


def _proj_pack(x, weight, src, etypes, dst):
    proj, packed = pl.pallas_call(
        _proj_pack_body,
        grid=(N // _BN,),
        in_specs=[
            pl.BlockSpec((_BN, D), lambda i: (i, 0)),
            pl.BlockSpec((R, D, D), lambda i: (0, 0, 0)),
            pl.BlockSpec((_E2, 128), lambda i: (0, 0)),
            pl.BlockSpec((_E2, 128), lambda i: (0, 0)),
            pl.BlockSpec((_E2, 128), lambda i: (0, 0)),
        ],
        out_specs=[
            pl.BlockSpec((_BN, R, D), lambda i: (i, 0, 0)),
            pl.BlockSpec((_E2, 128), lambda i: (0, 0)),
        ],
        out_shape=[
            jax.ShapeDtypeStruct((N, R, D), jnp.float32),
            jax.ShapeDtypeStruct((_E2, 128), jnp.int32),
        ],
    )(x, weight, src.reshape(_E2, 128), etypes.reshape(_E2, 128),
      dst.reshape(_E2, 128))
    return proj.reshape(N * R, D), packed.reshape(E)


def _combine_body(p_ref, b_ref, out_ref):
    out_ref[...] = p_ref[0] + p_ref[1] + b_ref[...]


def _combine(partials, h_bias):
    return pl.pallas_call(
        _combine_body,
        grid=(N // _BN,),
        in_specs=[
            pl.BlockSpec((NC, _BN, D), lambda i: (0, i, 0)),
            pl.BlockSpec((1, D), lambda i: (0, 0)),
        ],
        out_specs=pl.BlockSpec((_BN, D), lambda i: (i, 0)),
        out_shape=jax.ShapeDtypeStruct((N, D), jnp.float32),
    )(partials, h_bias.reshape(1, D))


@functools.cache
def _build_edge_kernel():
    mesh = plsc.VectorSubcoreMesh(core_axis_name="c", subcore_axis_name="s")
    return functools.partial(
        pl.kernel,
        mesh=mesh,
        out_type=jax.ShapeDtypeStruct((NC, N, D), jnp.float32),
        scratch_types=[
            pltpu.VMEM((4, C), jnp.int32),
            pltpu.VMEM((4, C), jnp.float32),
            pltpu.VMEM((4, C), jnp.int32),
            pltpu.VMEM((4, C), jnp.int32),
            pltpu.VMEM((C, D), jnp.float32),
            pltpu.VMEM((C, D), jnp.float32),
            pltpu.VMEM((C, D), jnp.float32),
            pltpu.VMEM((C, D), jnp.float32),
            pltpu.VMEM_SHARED((N, D), jnp.float32),
            pltpu.SemaphoreType.DMA((4,)),
            pltpu.SemaphoreType.DMA((4,)),
            pltpu.SemaphoreType.DMA((4,)),
            pltpu.SemaphoreType.DMA((4,)),
            pltpu.SemaphoreType.DMA,
        ],
    )(_edge_body)


def _edge_body(packed_hbm, norm_hbm, proj_hbm, zeros_hbm, out_hbm,
               packedc, normc, idxc, dstc, r0, r1, r2, r3, acc,
               pms, nms, gsem, ssem, zsem):
    c = lax.axis_index("c")
    s = lax.axis_index("s")
    w = s * NC + c
    rows = (r0, r1, r2, r3)
    ebase = w * EPW

    row0 = s * ROWS_PER_TILE
    zcopy = pltpu.make_async_copy(zeros_hbm.at[pl.ds(row0, ROWS_PER_TILE)],
                                  acc.at[pl.ds(row0, ROWS_PER_TILE)], zsem)
    zcopy.start()
    tail = pltpu.make_async_copy(zeros_hbm.at[pl.ds(TAIL_ROW0, TAIL_ROWS)],
                                 acc.at[pl.ds(TAIL_ROW0, TAIL_ROWS)], zsem)

    @pl.when(s == NS - 1)
    def _zero_tail():
        tail.start()


    def _meta_issue(t, slot):
        sl = pl.ds(ebase + t * C, C)
        pltpu.async_copy(packed_hbm.at[sl], packedc.at[slot], pms.at[slot])
        pltpu.async_copy(norm_hbm.at[sl], normc.at[slot], nms.at[slot])

    def _meta_wait(t, slot):
        sl = pl.ds(ebase + t * C, C)
        pltpu.make_async_copy(packed_hbm.at[sl], packedc.at[slot],
                              pms.at[slot]).wait()
        pltpu.make_async_copy(norm_hbm.at[sl], normc.at[slot],
                              nms.at[slot]).wait()

    def _unpack(slot):
        for g in range(C // L):
            sl = pl.ds(g * L, L)
            pv = packedc[slot, sl]
            idxc[slot, sl] = jax.lax.shift_right_logical(pv, DST_BITS)
            dstc[slot, sl] = jnp.bitwise_and(pv, (1 << DST_BITS) - 1)

    def _gather_issue(slot):
        pltpu.async_copy(proj_hbm.at[idxc.at[slot]], rows[slot],
                         gsem.at[slot])

    def _gather_wait(slot):
        pltpu.make_async_copy(proj_hbm.at[idxc.at[slot]], rows[slot],
                              gsem.at[slot]).wait()

    def _scale(slot):
        buf = rows[slot]

        def body(g, _):
            nv = normc[slot, pl.ds(g * L, L)]
            for l in range(L):
                sv = nv[l]
                e = g * L + l
                for k in range(D // L):
                    sk = pl.ds(k * L, L)
                    buf[e, sk] = buf[e, sk] * sv
            return 0
        lax.fori_loop(0, C // L, body, 0)

    def _scatter_issue(slot):
        pltpu.async_copy(rows[slot], acc.at[dstc.at[slot]], ssem.at[slot],
                         add=True)

    def _scatter_wait(slot):
        pltpu.make_async_copy(rows[slot], acc.at[dstc.at[slot]],
                              ssem.at[slot]).wait()

    with jax.named_scope("sc_init"):
        for t in range(4):
            _meta_issue(t, t)
        zcopy.wait()

        @pl.when(s == NS - 1)
        def _zero_tail_wait():
            tail.wait()
        plsc.subcore_barrier()

        for t in range(2):
            _meta_wait(t, t)
            _unpack(t)
            _gather_issue(t)

    def _step(t4, _):
        for i in range(4):
            t = 4 * t4 + i
            j = i
            j2 = (i + 2) % 4

            @pl.when(t < CH_PER_W)
            def _process():
                @pl.when(t + 2 < CH_PER_W)
                def _prefetch():
                    @pl.when(t >= 2)
                    def _drain_prev():
                        _scatter_wait(j2)
                    _meta_wait(t + 2, j2)
                    _unpack(j2)
                    _gather_issue(j2)
                _gather_wait(j)
                _scale(j)

                @pl.when(t + 4 < CH_PER_W)
                def _meta_next():
                    _meta_issue(t + 4, j)
                _scatter_issue(j)
        return 0

    with jax.named_scope("sc_main"):
        lax.fori_loop(0, (CH_PER_W + 3) // 4, _step, 0)

    with jax.named_scope("sc_flush"):
        for j in (1, 2, 3, 0):
            _scatter_wait(j)

        plsc.subcore_barrier()

        pltpu.sync_copy(acc.at[pl.ds(row0, ROWS_PER_TILE)],
                        out_hbm.at[c, pl.ds(row0, ROWS_PER_TILE)])

        @pl.when(s == NS - 1)
        def _copy_tail():
            pltpu.sync_copy(acc.at[pl.ds(TAIL_ROW0, TAIL_ROWS)],
                            out_hbm.at[c, pl.ds(TAIL_ROW0, TAIL_ROWS)])


def kernel(x, edge_index, etypes, norm, weight, h_bias):
    proj, packed = _proj_pack(x, weight, edge_index[0], etypes, edge_index[1])
    zeros = jnp.zeros((N, D), jnp.float32)
    partials = _build_edge_kernel()(packed, norm.reshape(E), proj, zeros)
    return _combine(partials, h_bias)


# speedup vs baseline: 1.0275x; 1.0275x over previous
import functools

import jax
import jax.numpy as jnp
from jax import lax
from jax.experimental import pallas as pl
from jax.experimental.pallas import tpu as pltpu
from jax.experimental.pallas import tpu_sc as plsc

N = 10000
E = 320000
D = 128
R = 8

NC = 2
NS = 16
NW = NC * NS
L = 16

C = 80
NCHUNK = E // C
CH_PER_W = NCHUNK // NW
EPW = CH_PER_W * C

ROWS_PER_TILE = 624
TAIL_ROW0 = NS * ROWS_PER_TILE
TAIL_ROWS = N - TAIL_ROW0

_BN = 1000


DST_BITS = 14
_E2 = E // 128


def _proj_pack_body(x_ref, w_ref, src_ref, et_ref, dst_ref,
                    proj_ref, packed_ref):
    xb = x_ref[...]
    for r in range(R):
        proj_ref[:, r, :] = jnp.dot(xb, w_ref[r],
                                    preferred_element_type=jnp.float32)

    @pl.when(pl.program_id(0) == 0)
    def _pack():
        idx = src_ref[...] * R + et_ref[...]
        packed_ref[...] = jnp.left_shift(idx, DST_BITS) + dst_ref[...]


def _proj_pack(x, weight, src, etypes, dst):
    proj, packed = pl.pallas_call(
        _proj_pack_body,
        grid=(N // _BN,),
        in_specs=[
            pl.BlockSpec((_BN, D), lambda i: (i, 0)),
            pl.BlockSpec((R, D, D), lambda i: (0, 0, 0)),
            pl.BlockSpec((_E2, 128), lambda i: (0, 0)),
            pl.BlockSpec((_E2, 128), lambda i: (0, 0)),
            pl.BlockSpec((_E2, 128), lambda i: (0, 0)),
        ],
        out_specs=[
            pl.BlockSpec((_BN, R, D), lambda i: (i, 0, 0)),
            pl.BlockSpec((_E2, 128), lambda i: (0, 0)),
        ],
        out_shape=[
            jax.ShapeDtypeStruct((N, R, D), jnp.float32),
            jax.ShapeDtypeStruct((_E2, 128), jnp.int32),
        ],
    )(x, weight, src.reshape(_E2, 128), etypes.reshape(_E2, 128),
      dst.reshape(_E2, 128))
    return proj.reshape(N * R, D), packed.reshape(E)


def _combine_body(p_ref, b_ref, out_ref):
    out_ref[...] = p_ref[0] + p_ref[1] + b_ref[...]


def _combine(partials, h_bias):
    return pl.pallas_call(
        _combine_body,
        grid=(N // _BN,),
        in_specs=[
            pl.BlockSpec((NC, _BN, D), lambda i: (0, i, 0)),
            pl.BlockSpec((1, D), lambda i: (0, 0)),
        ],
        out_specs=pl.BlockSpec((_BN, D), lambda i: (i, 0)),
        out_shape=jax.ShapeDtypeStruct((N, D), jnp.float32),
    )(partials, h_bias.reshape(1, D))


@functools.cache
def _build_edge_kernel():
    mesh = plsc.VectorSubcoreMesh(core_axis_name="c", subcore_axis_name="s")
    return functools.partial(
        pl.kernel,
        mesh=mesh,
        out_type=jax.ShapeDtypeStruct((NC, N, D), jnp.float32),
        scratch_types=[
            pltpu.VMEM((4, C), jnp.int32),
            pltpu.VMEM((4, C), jnp.float32),
            pltpu.VMEM((4, C), jnp.int32),
            pltpu.VMEM((4, C), jnp.int32),
            pltpu.VMEM((C, D), jnp.float32),
            pltpu.VMEM((C, D), jnp.float32),
            pltpu.VMEM((C, D), jnp.float32),
            pltpu.VMEM((C, D), jnp.float32),
            pltpu.VMEM_SHARED((N, D), jnp.float32),
            pltpu.SemaphoreType.DMA((4,)),
            pltpu.SemaphoreType.DMA((4,)),
            pltpu.SemaphoreType.DMA((4,)),
            pltpu.SemaphoreType.DMA((4,)),
            pltpu.SemaphoreType.DMA,
        ],
    )(_edge_body)


def _edge_body(packed_hbm, norm_hbm, proj_hbm, out_hbm,
               packedc, normc, idxc, dstc, r0, r1, r2, r3, acc,
               pms, nms, gsem, ssem, zsem):
    c = lax.axis_index("c")
    s = lax.axis_index("s")
    w = s * NC + c
    rows = (r0, r1, r2, r3)
    ebase = w * EPW

    def _zero_rows(e, _):
        for k in range(D // L):
            r0[e, pl.ds(k * L, L)] = jnp.zeros((L,), jnp.float32)
        return 0
    lax.fori_loop(0, C, _zero_rows, 0)
    row0 = s * ROWS_PER_TILE
    _NZ = ROWS_PER_TILE // C
    _ZREM = ROWS_PER_TILE - _NZ * C
    zcopies = [
        pltpu.make_async_copy(r0, acc.at[pl.ds(row0 + j * C, C)], zsem)
        for j in range(_NZ)
    ] + [
        pltpu.make_async_copy(r0.at[pl.ds(0, _ZREM)],
                              acc.at[pl.ds(row0 + _NZ * C, _ZREM)], zsem)
    ]
    for zc in zcopies:
        zc.start()
    tail = pltpu.make_async_copy(r0.at[pl.ds(0, TAIL_ROWS)],
                                 acc.at[pl.ds(TAIL_ROW0, TAIL_ROWS)], zsem)

    @pl.when(s == NS - 1)
    def _zero_tail():
        tail.start()


    def _meta_issue(t, slot):
        sl = pl.ds(ebase + t * C, C)
        pltpu.async_copy(packed_hbm.at[sl], packedc.at[slot], pms.at[slot])
        pltpu.async_copy(norm_hbm.at[sl], normc.at[slot], nms.at[slot])

    def _meta_wait(t, slot):
        sl = pl.ds(ebase + t * C, C)
        pltpu.make_async_copy(packed_hbm.at[sl], packedc.at[slot],
                              pms.at[slot]).wait()
        pltpu.make_async_copy(norm_hbm.at[sl], normc.at[slot],
                              nms.at[slot]).wait()

    def _unpack(slot):
        for g in range(C // L):
            sl = pl.ds(g * L, L)
            pv = packedc[slot, sl]
            idxc[slot, sl] = jax.lax.shift_right_logical(pv, DST_BITS)
            dstc[slot, sl] = jnp.bitwise_and(pv, (1 << DST_BITS) - 1)

    def _gather_issue(slot):
        pltpu.async_copy(proj_hbm.at[idxc.at[slot]], rows[slot],
                         gsem.at[slot])

    def _gather_wait(slot):
        pltpu.make_async_copy(proj_hbm.at[idxc.at[slot]], rows[slot],
                              gsem.at[slot]).wait()

    def _scale(slot):
        buf = rows[slot]

        def body(g, _):
            nv = normc[slot, pl.ds(g * L, L)]
            for l in range(L):
                sv = nv[l]
                e = g * L + l
                for k in range(D // L):
                    sk = pl.ds(k * L, L)
                    buf[e, sk] = buf[e, sk] * sv
            return 0
        lax.fori_loop(0, C // L, body, 0)

    def _scatter_issue(slot):
        pltpu.async_copy(rows[slot], acc.at[dstc.at[slot]], ssem.at[slot],
                         add=True)

    def _scatter_wait(slot):
        pltpu.make_async_copy(rows[slot], acc.at[dstc.at[slot]],
                              ssem.at[slot]).wait()

    with jax.named_scope("sc_init"):
        for t in range(4):
            _meta_issue(t, t)
        for zc in zcopies:
            zc.wait()

        @pl.when(s == NS - 1)
        def _zero_tail_wait():
            tail.wait()
        plsc.subcore_barrier()

        for t in range(2):
            _meta_wait(t, t)
            _unpack(t)
            _gather_issue(t)

    def _step(t4, _):
        for i in range(4):
            t = 4 * t4 + i
            j = i
            j2 = (i + 2) % 4

            @pl.when(t < CH_PER_W)
            def _process():
                @pl.when(t + 2 < CH_PER_W)
                def _prefetch():
                    @pl.when(t >= 2)
                    def _drain_prev():
                        _scatter_wait(j2)
                    _meta_wait(t + 2, j2)
                    _unpack(j2)
                    _gather_issue(j2)
                _gather_wait(j)
                _scale(j)

                @pl.when(t + 4 < CH_PER_W)
                def _meta_next():
                    _meta_issue(t + 4, j)
                _scatter_issue(j)
        return 0

    with jax.named_scope("sc_main"):
        lax.fori_loop(0, (CH_PER_W + 3) // 4, _step, 0)

    with jax.named_scope("sc_flush"):
        for j in (1, 2, 3, 0):
            _scatter_wait(j)

        plsc.subcore_barrier()

        pltpu.sync_copy(acc.at[pl.ds(row0, ROWS_PER_TILE)],
                        out_hbm.at[c, pl.ds(row0, ROWS_PER_TILE)])

        @pl.when(s == NS - 1)
        def _copy_tail():
            pltpu.sync_copy(acc.at[pl.ds(TAIL_ROW0, TAIL_ROWS)],
                            out_hbm.at[c, pl.ds(TAIL_ROW0, TAIL_ROWS)])


def kernel(x, edge_index, etypes, norm, weight, h_bias):
    proj, packed = _proj_pack(x, weight, edge_index[0], etypes, edge_index[1])
    partials = _build_edge_kernel()(packed, norm.reshape(E), proj)
    return _combine(partials, h_bias)


# speedup vs baseline: 1.0280x; 1.0006x over previous
import functools

import jax
import jax.numpy as jnp
from jax import lax
from jax.experimental import pallas as pl
from jax.experimental.pallas import tpu as pltpu
from jax.experimental.pallas import tpu_sc as plsc

N = 10000
E = 320000
D = 128
R = 8

NC = 2
NS = 16
NW = NC * NS
L = 16

C = 80
NCHUNK = E // C
CH_PER_W = NCHUNK // NW
EPW = CH_PER_W * C

ROWS_PER_TILE = 624
TAIL_ROW0 = NS * ROWS_PER_TILE
TAIL_ROWS = N - TAIL_ROW0

_BN = 1000

DST_BITS = 14
_E2 = E // 128


def _proj_pack_body(x_ref, w_ref, src_ref, et_ref, dst_ref,
                    proj_ref, packed_ref):
    xb = x_ref[...]
    for r in range(R):
        proj_ref[:, r, :] = jnp.dot(xb, w_ref[r],
                                    preferred_element_type=jnp.float32)

    @pl.when(pl.program_id(0) == 0)
    def _pack():
        idx = src_ref[...] * R + et_ref[...]
        packed_ref[...] = jnp.left_shift(idx, DST_BITS) + dst_ref[...]


def _proj_pack(x, weight, src, etypes, dst):
    proj, packed = pl.pallas_call(
        _proj_pack_body,
        grid=(N // _BN,),
        in_specs=[
            pl.BlockSpec((_BN, D), lambda i: (i, 0)),
            pl.BlockSpec((R, D, D), lambda i: (0, 0, 0)),
            pl.BlockSpec((_E2, 128), lambda i: (0, 0)),
            pl.BlockSpec((_E2, 128), lambda i: (0, 0)),
            pl.BlockSpec((_E2, 128), lambda i: (0, 0)),
        ],
        out_specs=[
            pl.BlockSpec((_BN, R, D), lambda i: (i, 0, 0)),
            pl.BlockSpec((_E2, 128), lambda i: (0, 0)),
        ],
        out_shape=[
            jax.ShapeDtypeStruct((N, R, D), jnp.float32),
            jax.ShapeDtypeStruct((_E2, 128), jnp.int32),
        ],
    )(x, weight, src.reshape(_E2, 128), etypes.reshape(_E2, 128),
      dst.reshape(_E2, 128))
    return proj.reshape(N * R, D), packed.reshape(E)


def _combine_body(p_ref, b_ref, out_ref):
    out_ref[...] = p_ref[0] + p_ref[1] + b_ref[...]


def _combine(partials, h_bias):
    return pl.pallas_call(
        _combine_body,
        grid=(N // _BN,),
        in_specs=[
            pl.BlockSpec((NC, _BN, D), lambda i: (0, i, 0)),
            pl.BlockSpec((1, D), lambda i: (0, 0)),
        ],
        out_specs=pl.BlockSpec((_BN, D), lambda i: (i, 0)),
        out_shape=jax.ShapeDtypeStruct((N, D), jnp.float32),
    )(partials, h_bias.reshape(1, D))


@functools.cache
def _build_edge_kernel():
    mesh = plsc.VectorSubcoreMesh(core_axis_name="c", subcore_axis_name="s")
    return functools.partial(
        pl.kernel,
        mesh=mesh,
        out_type=jax.ShapeDtypeStruct((NC, N, D), jnp.float32),
        scratch_types=[
            pltpu.VMEM((4, C), jnp.int32),
            pltpu.VMEM((4, C), jnp.float32),
            pltpu.VMEM((4, C), jnp.int32),
            pltpu.VMEM((4, C), jnp.int32),
            pltpu.VMEM((C, D), jnp.float32),
            pltpu.VMEM((C, D), jnp.float32),
            pltpu.VMEM((C, D), jnp.float32),
            pltpu.VMEM((C, D), jnp.float32),
            pltpu.VMEM_SHARED((N, D), jnp.float32),
            pltpu.SemaphoreType.DMA((4,)),
            pltpu.SemaphoreType.DMA((4,)),
            pltpu.SemaphoreType.DMA((4,)),
            pltpu.SemaphoreType.DMA((4,)),
            pltpu.SemaphoreType.DMA,
        ],
    )(_edge_body)


def _edge_body(packed_hbm, norm_hbm, proj_hbm, out_hbm,
               packedc, normc, idxc, dstc, r0, r1, r2, r3, acc,
               pms, nms, gsem, ssem, zsem):
    c = lax.axis_index("c")
    s = lax.axis_index("s")
    w = s * NC + c
    rows = (r0, r1, r2, r3)
    ebase = w * EPW

    def _zero_rows(e, _):
        for k in range(D // L):
            r0[e, pl.ds(k * L, L)] = jnp.zeros((L,), jnp.float32)
        return 0
    lax.fori_loop(0, C, _zero_rows, 0)
    row0 = s * ROWS_PER_TILE
    _NZ = ROWS_PER_TILE // C
    _ZREM = ROWS_PER_TILE - _NZ * C
    zcopies = [
        pltpu.make_async_copy(r0, acc.at[pl.ds(row0 + j * C, C)], zsem)
        for j in range(_NZ)
    ] + [
        pltpu.make_async_copy(r0.at[pl.ds(0, _ZREM)],
                              acc.at[pl.ds(row0 + _NZ * C, _ZREM)], zsem)
    ]
    for zc in zcopies:
        zc.start()
    tail = pltpu.make_async_copy(r0.at[pl.ds(0, TAIL_ROWS)],
                                 acc.at[pl.ds(TAIL_ROW0, TAIL_ROWS)], zsem)

    @pl.when(s == NS - 1)
    def _zero_tail():
        tail.start()


    def _meta_issue(t, slot):
        sl = pl.ds(ebase + t * C, C)
        pltpu.async_copy(packed_hbm.at[sl], packedc.at[slot], pms.at[slot])
        pltpu.async_copy(norm_hbm.at[sl], normc.at[slot], nms.at[slot])

    def _meta_wait(t, slot):
        sl = pl.ds(ebase + t * C, C)
        pltpu.make_async_copy(packed_hbm.at[sl], packedc.at[slot],
                              pms.at[slot]).wait()
        pltpu.make_async_copy(norm_hbm.at[sl], normc.at[slot],
                              nms.at[slot]).wait()

    def _unpack(slot):
        for g in range(C // L):
            sl = pl.ds(g * L, L)
            pv = packedc[slot, sl]
            idxc[slot, sl] = jax.lax.shift_right_logical(pv, DST_BITS)
            dstc[slot, sl] = jnp.bitwise_and(pv, (1 << DST_BITS) - 1)

    def _gather_issue(slot):
        pltpu.async_copy(proj_hbm.at[idxc.at[slot]], rows[slot],
                         gsem.at[slot])

    def _gather_wait(slot):
        pltpu.make_async_copy(proj_hbm.at[idxc.at[slot]], rows[slot],
                              gsem.at[slot]).wait()

    def _scale(slot):
        buf = rows[slot]

        def body(g, _):
            nv = normc[slot, pl.ds(g * L, L)]
            for l in range(L):
                sv = nv[l]
                e = g * L + l
                for k in range(D // L):
                    sk = pl.ds(k * L, L)
                    buf[e, sk] = buf[e, sk] * sv
            return 0
        lax.fori_loop(0, C // L, body, 0)

    def _scatter_issue(slot):
        pltpu.async_copy(rows[slot], acc.at[dstc.at[slot]], ssem.at[slot],
                         add=True)

    def _scatter_wait(slot):
        pltpu.make_async_copy(rows[slot], acc.at[dstc.at[slot]],
                              ssem.at[slot]).wait()

    for t in range(4):
        _meta_issue(t, t)
    for zc in zcopies:
        zc.wait()

    @pl.when(s == NS - 1)
    def _zero_tail_wait():
        tail.wait()
    plsc.subcore_barrier()

    for t in range(2):
        _meta_wait(t, t)
        _unpack(t)
        _gather_issue(t)

    def _step(t4, _):
        for i in range(4):
            t = 4 * t4 + i
            j = i
            j2 = (i + 2) % 4

            @pl.when(t < CH_PER_W)
            def _process():
                @pl.when(t + 2 < CH_PER_W)
                def _prefetch():
                    @pl.when(t >= 2)
                    def _drain_prev():
                        _scatter_wait(j2)
                    _meta_wait(t + 2, j2)
                    _unpack(j2)
                    _gather_issue(j2)
                _gather_wait(j)
                _scale(j)

                @pl.when(t + 4 < CH_PER_W)
                def _meta_next():
                    _meta_issue(t + 4, j)
                _scatter_issue(j)
        return 0

    lax.fori_loop(0, (CH_PER_W + 3) // 4, _step, 0)

    for j in (1, 2, 3, 0):
        _scatter_wait(j)

    plsc.subcore_barrier()

    pltpu.sync_copy(acc.at[pl.ds(row0, ROWS_PER_TILE)],
                    out_hbm.at[c, pl.ds(row0, ROWS_PER_TILE)])

    @pl.when(s == NS - 1)
    def _copy_tail():
        pltpu.sync_copy(acc.at[pl.ds(TAIL_ROW0, TAIL_ROWS)],
                        out_hbm.at[c, pl.ds(TAIL_ROW0, TAIL_ROWS)])


def kernel(x, edge_index, etypes, norm, weight, h_bias):
    proj, packed = _proj_pack(x, weight, edge_index[0], etypes,
                              edge_index[1])
    partials = _build_edge_kernel()(packed, norm.reshape(E), proj)
    return _combine(partials, h_bias)
